# baseline (speedup 1.0000x reference)
import jax
import jax.numpy as jnp
from jax import lax
from jax.experimental import pallas as pl
from jax.experimental.pallas import tpu as pltpu

_K = 5


def _acc_body(gt_sm_ref, gt_ref, pred_ref, out_ref):
    i = pl.program_id(0)
    pred = pred_ref[...]
    g = gt_ref[...]
    rb, n = pred.shape
    sub_iota = lax.broadcasted_iota(jnp.int32, (8, 128), 0)
    lane_iota = lax.broadcasted_iota(jnp.int32, (8, 128), 1)
    row_iota = lax.broadcasted_iota(jnp.int32, (rb, 1), 0)

    v = jnp.zeros((rb, 1), jnp.float32)
    for r in range(rb):
        gr = gt_sm_ref[r, 0]
        cb = pl.multiple_of((gr // 128) * 128, 128)
        tile = pred_ref[pl.ds((r // 8) * 8, 8), pl.ds(cb, 128)]
        val = jnp.sum(jnp.where((sub_iota == r % 8) & (lane_iota == gr % 128),
                                tile, 0.0))
        v = v + jnp.where(row_iota == r, val, 0.0)

    cnt_gt = jnp.sum((pred > v).astype(jnp.int32), axis=1)
    cnt_eq = jnp.sum((pred == v).astype(jnp.int32), axis=1)

    @pl.when(i == 0)
    def _():
        out_ref[...] = jnp.zeros((1, 1), jnp.float32)

    ambiguous = jnp.any((cnt_gt < _K) & (cnt_gt + cnt_eq - 1 >= _K))

    @pl.when(jnp.logical_not(ambiguous))
    def _():
        part = jnp.sum((cnt_gt < _K).astype(jnp.float32)).reshape(1, 1)
        out_ref[...] += part

    @pl.when(ambiguous)
    def _():
        col = lax.broadcasted_iota(jnp.int32, (rb, n), 1)
        cnt_eq_low = jnp.sum(((pred == v) & (col < g)).astype(jnp.int32),
                             axis=1)
        part = jnp.sum(((cnt_gt + cnt_eq_low) < _K)
                       .astype(jnp.float32)).reshape(1, 1)
        out_ref[...] += part


def kernel(pred, gt):
    b, n = pred.shape
    rb = 32
    grid = (b // rb,)
    gt2 = gt.reshape(b, 1)
    out = pl.pallas_call(
        _acc_body,
        grid=grid,
        in_specs=[
            pl.BlockSpec((rb, 1), lambda i: (i, 0),
                         memory_space=pltpu.SMEM),
            pl.BlockSpec((rb, 1), lambda i: (i, 0)),
            pl.BlockSpec((rb, n), lambda i: (i, 0)),
        ],
        out_specs=pl.BlockSpec((1, 1), lambda i: (0, 0)),
        out_shape=jax.ShapeDtypeStruct((1, 1), jnp.float32),
    )(gt2, gt2, pred)
    return out[0, 0] / b


# speedup vs baseline: 1.0442x; 1.0442x over previous
import jax
import jax.numpy as jnp
from jax import lax
from jax.experimental import pallas as pl
from jax.experimental.pallas import tpu as pltpu

_K = 5


def _acc_body(gt_sm_ref, gt_ref, pred_ref, out_ref):
    i = pl.program_id(0)
    pred = pred_ref[...]
    g = gt_ref[...]
    rb, n = pred.shape
    sub_iota = lax.broadcasted_iota(jnp.int32, (8, 128), 0)
    lane_iota = lax.broadcasted_iota(jnp.int32, (8, 128), 1)
    row_iota = lax.broadcasted_iota(jnp.int32, (rb, 1), 0)

    v = jnp.zeros((rb, 1), jnp.float32)
    for r in range(rb):
        gr = gt_sm_ref[r, 0]
        cb = pl.multiple_of((gr // 128) * 128, 128)
        tile = pred_ref[pl.ds((r // 8) * 8, 8), pl.ds(cb, 128)]
        val = jnp.sum(jnp.where((sub_iota == r % 8) & (lane_iota == gr % 128),
                                tile, 0.0))
        v = v + jnp.where(row_iota == r, val, 0.0)

    cnt_gt = jnp.sum((pred > v).astype(jnp.int32), axis=1)
    cnt_eq = jnp.sum((pred == v).astype(jnp.int32), axis=1)

    @pl.when(i == 0)
    def _():
        out_ref[...] = jnp.zeros((1, 1), jnp.float32)

    ambiguous = jnp.any((cnt_gt < _K) & (cnt_gt + cnt_eq - 1 >= _K))

    @pl.when(jnp.logical_not(ambiguous))
    def _():
        part = jnp.sum((cnt_gt < _K).astype(jnp.float32)).reshape(1, 1)
        out_ref[...] += part

    @pl.when(ambiguous)
    def _():
        col = lax.broadcasted_iota(jnp.int32, (rb, n), 1)
        cnt_eq_low = jnp.sum(((pred == v) & (col < g)).astype(jnp.int32),
                             axis=1)
        part = jnp.sum(((cnt_gt + cnt_eq_low) < _K)
                       .astype(jnp.float32)).reshape(1, 1)
        out_ref[...] += part


def kernel(pred, gt):
    b, n = pred.shape
    rb = 64
    grid = (b // rb,)
    gt2 = gt.reshape(b, 1)
    out = pl.pallas_call(
        _acc_body,
        grid=grid,
        in_specs=[
            pl.BlockSpec((rb, 1), lambda i: (i, 0),
                         memory_space=pltpu.SMEM),
            pl.BlockSpec((rb, 1), lambda i: (i, 0)),
            pl.BlockSpec((rb, n), lambda i: (i, 0)),
        ],
        out_specs=pl.BlockSpec((1, 1), lambda i: (0, 0)),
        out_shape=jax.ShapeDtypeStruct((1, 1), jnp.float32),
    )(gt2, gt2, pred)
    return out[0, 0] / b
